# baseline (speedup 1.0000x reference)
import jax
import jax.numpy as jnp
from jax import lax
from jax.experimental import pallas as pl
from jax.experimental.pallas import tpu as pltpu

B, T, R = 16, 100, 200
D, H, DH, FF = 128, 4, 32, 2048
HID, V, S, NQ, NL = 4096, 32000, 512, 8, 2
EPS = 1e-5


def _shift_prev(x):
    return jnp.concatenate([jnp.zeros_like(x[:, :1, :]), x[:, :-1, :]], axis=1)


def _shift_next(x):
    return jnp.concatenate([x[:, 1:, :], jnp.zeros_like(x[:, :1, :])], axis=1)


def _conv_bn_kernel(bold_ref, w1a_ref, w1b_ref, b1a_ref, b1b_ref,
                    g1a_ref, g1b_ref, h1a_ref, h1b_ref,
                    w2a_ref, w2b_ref, c2b_ref, g2_ref, h2_ref, out_ref):
    x = jnp.nan_to_num(bold_ref[...])
    xp, xn = _shift_prev(x), _shift_next(x)
    a = w1a_ref[0] * xp + w1a_ref[1] * x + w1a_ref[2] * xn + b1a_ref[...]
    b = w1b_ref[0] * xp + w1b_ref[1] * x + w1b_ref[2] * xn + b1b_ref[...]

    def bn(y, g, h):
        m = jnp.mean(y, axis=(0, 1), keepdims=True)
        v = jnp.mean((y - m) ** 2, axis=(0, 1), keepdims=True)
        return (y - m) * lax.rsqrt(v + EPS) * g + h

    a = jnp.maximum(bn(a, g1a_ref[...], h1a_ref[...]), 0.0)
    b = jnp.maximum(bn(b, g1b_ref[...], h1b_ref[...]), 0.0)
    ap, an = _shift_prev(a), _shift_next(a)
    bp, bnx = _shift_prev(b), _shift_next(b)
    y = (w2a_ref[0] * ap + w2a_ref[1] * a + w2a_ref[2] * an
         + w2b_ref[0] * bp + w2b_ref[1] * b + w2b_ref[2] * bnx + c2b_ref[...])
    out_ref[...] = jnp.maximum(bn(y, g2_ref[...], h2_ref[...]), 0.0)


def _ln(x, g, h):
    m = jnp.mean(x, axis=-1, keepdims=True)
    v = jnp.mean((x - m) ** 2, axis=-1, keepdims=True)
    return (x - m) * lax.rsqrt(v + EPS) * g + h


def _softmax(x):
    m = jnp.max(x, axis=-1, keepdims=True)
    e = jnp.exp(x - m)
    return e / jnp.sum(e, axis=-1, keepdims=True)


def _dot_t(x, w):
    return lax.dot_general(x.astype(jnp.bfloat16), w.astype(jnp.bfloat16),
                           (((1,), (1,)), ((), ())),
                           preferred_element_type=jnp.float32)


def _dot(x, w):
    return lax.dot_general(x.astype(jnp.bfloat16), w.astype(jnp.bfloat16),
                           (((1,), (0,)), ((), ())),
                           preferred_element_type=jnp.float32)


def _encoder_kernel(x2_ref, tpw_ref, tpb_ref, wq_ref, wqb_ref, wk_ref, wkb_ref,
                    ln1g_ref, ln1b_ref, qkvw_ref, qkvb_ref, outw_ref, outb_ref,
                    ln2g_ref, ln2b_ref, ff1w_ref, ff1b_ref, ff2w_ref, ff2b_ref,
                    pw_ref, pb_ref, lg_ref, lb_ref, qt_ref, o_ref):
    x2 = x2_ref[0]
    h = lax.dot_general(x2.astype(jnp.bfloat16),
                        tpw_ref[...].astype(jnp.bfloat16),
                        (((0,), (1,)), ((), ())),
                        preferred_element_type=jnp.float32) + tpb_ref[...]
    q = _dot_t(h, wq_ref[...]) + wqb_ref[...]
    k = _dot_t(h, wk_ref[...]) + wkb_ref[...]
    adj = _softmax(_dot_t(q, k) * (D ** -0.5))
    z = _dot(adj, h)
    for l in range(NL):
        y = _ln(z, ln1g_ref[l], ln1b_ref[l])
        qkv = _dot_t(y, qkvw_ref[l]) + qkvb_ref[l]
        qh, kh, vh = qkv[:, :D], qkv[:, D:2 * D], qkv[:, 2 * D:]
        outs = []
        for hh in range(H):
            sl = slice(hh * DH, (hh + 1) * DH)
            s = _dot_t(qh[:, sl], kh[:, sl]) * (DH ** -0.5)
            outs.append(_dot(_softmax(s), vh[:, sl]))
        o = jnp.concatenate(outs, axis=1)
        z = z + _dot_t(o, outw_ref[l]) + outb_ref[l]
        y2 = _ln(z, ln2g_ref[l], ln2b_ref[l])
        f = jnp.maximum(_dot_t(y2, ff1w_ref[l]) + ff1b_ref[l], 0.0)
        z = z + _dot_t(f, ff2w_ref[l]) + ff2b_ref[l]
    qt = qt_ref[0]
    att = _softmax(_dot_t(qt, z) * (D ** -0.5))
    ctx = _dot(att, z)
    br = _dot_t(ctx, pw_ref[...]) + pb_ref[...]
    o_ref[0] = _ln(br, lg_ref[...], lb_ref[...])


def _gather_kernel(ids_ref, brain_ref, table_ref, out_ref, sem_r):
    b = pl.program_id(0)
    out_ref[0, :NQ, :] = brain_ref[0]
    for s in range(S):
        tok = ids_ref[b, s]
        pltpu.make_async_copy(
            table_ref.at[tok], out_ref.at[0, NQ + s], sem_r).start()
    wait_cp = pltpu.make_async_copy(
        table_ref.at[0], out_ref.at[0, NQ], sem_r)
    for s in range(S):
        wait_cp.wait()


def kernel(bold, input_ids, attention_mask, labels,
           conv1_w, conv1_b, bn1_g, bn1_b, conv2_w, conv2_b, bn2_g, bn2_b,
           tproj_w, tproj_b, wq_w, wq_b, wk_w, wk_b,
           enc_ln1_g, enc_ln1_b, enc_qkv_w, enc_qkv_b, enc_out_w, enc_out_b,
           enc_ln2_g, enc_ln2_b, enc_ff1_w, enc_ff1_b, enc_ff2_w, enc_ff2_b,
           proj_w, proj_b, lnf_g, lnf_b, query_tokens, embed_table):
    f32 = jnp.float32
    w1 = conv1_w.reshape(R, 2, 3)
    w1a = w1[:, 0, :].transpose(1, 0).reshape(3, 1, R)
    w1b = w1[:, 1, :].transpose(1, 0).reshape(3, 1, R)
    b1 = conv1_b.reshape(R, 2)
    b1a, b1b = b1[:, 0].reshape(1, 1, R), b1[:, 1].reshape(1, 1, R)
    g1 = bn1_g.reshape(R, 2)
    g1a, g1b = g1[:, 0].reshape(1, 1, R), g1[:, 1].reshape(1, 1, R)
    h1 = bn1_b.reshape(R, 2)
    h1a, h1b = h1[:, 0].reshape(1, 1, R), h1[:, 1].reshape(1, 1, R)
    w2a = conv2_w[:, 0, :].transpose(1, 0).reshape(3, 1, R)
    w2b = conv2_w[:, 1, :].transpose(1, 0).reshape(3, 1, R)
    c2b = conv2_b.reshape(1, 1, R)
    g2, h2 = bn2_g.reshape(1, 1, R), bn2_b.reshape(1, 1, R)

    x2 = pl.pallas_call(
        _conv_bn_kernel,
        out_shape=jax.ShapeDtypeStruct((B, T, R), f32),
        name="conv_bn",
    )(bold, w1a, w1b, b1a, b1b, g1a, g1b, h1a, h1b, w2a, w2b, c2b, g2, h2)

    full = lambda shape: pl.BlockSpec(shape, lambda b: (0,) * len(shape))
    enc_in_specs = [
        pl.BlockSpec((1, T, R), lambda b: (b, 0, 0)),
        full((D, T)), full((1, D)),
        full((D, D)), full((1, D)), full((D, D)), full((1, D)),
        full((NL, D)), full((NL, D)),
        full((NL, 3 * D, D)), full((NL, 3 * D)),
        full((NL, D, D)), full((NL, D)),
        full((NL, D)), full((NL, D)),
        full((NL, FF, D)), full((NL, FF)),
        full((NL, D, FF)), full((NL, D)),
        full((HID, D)), full((1, HID)), full((1, HID)), full((1, HID)),
        full((1, NQ, D)),
    ]
    brain = pl.pallas_call(
        _encoder_kernel,
        grid=(B,),
        in_specs=enc_in_specs,
        out_specs=pl.BlockSpec((1, NQ, HID), lambda b: (b, 0, 0)),
        out_shape=jax.ShapeDtypeStruct((B, NQ, HID), f32),
        compiler_params=pltpu.CompilerParams(
            dimension_semantics=("parallel",),
        ),
        name="encoder",
    )(x2, tproj_w, tproj_b.reshape(1, D),
      wq_w, wq_b.reshape(1, D), wk_w, wk_b.reshape(1, D),
      enc_ln1_g, enc_ln1_b, enc_qkv_w, enc_qkv_b, enc_out_w, enc_out_b,
      enc_ln2_g, enc_ln2_b, enc_ff1_w, enc_ff1_b, enc_ff2_w, enc_ff2_b,
      proj_w, proj_b.reshape(1, HID), lnf_g.reshape(1, HID),
      lnf_b.reshape(1, HID), query_tokens)

    inputs_embeds = pl.pallas_call(
        _gather_kernel,
        grid_spec=pltpu.PrefetchScalarGridSpec(
            num_scalar_prefetch=1,
            grid=(B,),
            in_specs=[pl.BlockSpec((1, NQ, HID), lambda b, ids: (b, 0, 0)),
                      pl.BlockSpec(memory_space=pl.ANY)],
            out_specs=pl.BlockSpec((1, NQ + S, HID), lambda b, ids: (b, 0, 0)),
            scratch_shapes=[pltpu.SemaphoreType.DMA],
        ),
        out_shape=jax.ShapeDtypeStruct((B, NQ + S, HID), f32),
        compiler_params=pltpu.CompilerParams(
            dimension_semantics=("arbitrary",),
            vmem_limit_bytes=50 * 1024 * 1024,
        ),
        name="gather_concat",
    )(input_ids.astype(jnp.int32), brain, embed_table)

    full_mask = jnp.concatenate(
        [jnp.ones((B, NQ), attention_mask.dtype), attention_mask], axis=1)
    full_labels = jnp.concatenate(
        [jnp.full((B, NQ), -100, labels.dtype), labels], axis=1)
    return inputs_embeds, full_mask, full_labels


# speedup vs baseline: 1.0231x; 1.0231x over previous
import jax
import jax.numpy as jnp
from jax import lax
from jax.experimental import pallas as pl
from jax.experimental.pallas import tpu as pltpu

B, T, R = 16, 100, 200
D, H, DH, FF = 128, 4, 32, 2048
HID, V, S, NQ, NL = 4096, 32000, 512, 8, 2
EPS = 1e-5


def _shift_prev(x):
    return jnp.concatenate([jnp.zeros_like(x[:, :1, :]), x[:, :-1, :]], axis=1)


def _shift_next(x):
    return jnp.concatenate([x[:, 1:, :], jnp.zeros_like(x[:, :1, :])], axis=1)


def _conv_bn_kernel(bold_ref, w1a_ref, w1b_ref, b1a_ref, b1b_ref,
                    g1a_ref, g1b_ref, h1a_ref, h1b_ref,
                    w2a_ref, w2b_ref, c2b_ref, g2_ref, h2_ref, out_ref):
    x = jnp.nan_to_num(bold_ref[...])
    xp, xn = _shift_prev(x), _shift_next(x)
    a = w1a_ref[0] * xp + w1a_ref[1] * x + w1a_ref[2] * xn + b1a_ref[...]
    b = w1b_ref[0] * xp + w1b_ref[1] * x + w1b_ref[2] * xn + b1b_ref[...]

    def bn(y, g, h):
        m = jnp.mean(y, axis=(0, 1), keepdims=True)
        v = jnp.mean((y - m) ** 2, axis=(0, 1), keepdims=True)
        return (y - m) * lax.rsqrt(v + EPS) * g + h

    a = jnp.maximum(bn(a, g1a_ref[...], h1a_ref[...]), 0.0)
    b = jnp.maximum(bn(b, g1b_ref[...], h1b_ref[...]), 0.0)
    ap, an = _shift_prev(a), _shift_next(a)
    bp, bnx = _shift_prev(b), _shift_next(b)
    y = (w2a_ref[0] * ap + w2a_ref[1] * a + w2a_ref[2] * an
         + w2b_ref[0] * bp + w2b_ref[1] * b + w2b_ref[2] * bnx + c2b_ref[...])
    out_ref[...] = jnp.maximum(bn(y, g2_ref[...], h2_ref[...]), 0.0)


def _ln(x, g, h):
    m = jnp.mean(x, axis=-1, keepdims=True)
    v = jnp.mean((x - m) ** 2, axis=-1, keepdims=True)
    return (x - m) * lax.rsqrt(v + EPS) * g + h


def _softmax(x):
    m = jnp.max(x, axis=-1, keepdims=True)
    e = jnp.exp(x - m)
    return e / jnp.sum(e, axis=-1, keepdims=True)


def _dot_t(x, w):
    return lax.dot_general(x.astype(jnp.bfloat16), w.astype(jnp.bfloat16),
                           (((1,), (1,)), ((), ())),
                           preferred_element_type=jnp.float32)


def _dot(x, w):
    return lax.dot_general(x.astype(jnp.bfloat16), w.astype(jnp.bfloat16),
                           (((1,), (0,)), ((), ())),
                           preferred_element_type=jnp.float32)


_EG = 4


def _encoder_kernel(x2_ref, tpw_ref, tpb_ref, wq_ref, wqb_ref, wk_ref, wkb_ref,
                    ln1g_ref, ln1b_ref, qkvw_ref, qkvb_ref, outw_ref, outb_ref,
                    ln2g_ref, ln2b_ref, ff1w_ref, ff1b_ref, ff2w_ref, ff2b_ref,
                    pw_ref, pb_ref, lg_ref, lb_ref, qt_ref, o_ref):
    for g in range(_EG):
        _encoder_one(x2_ref[g], tpw_ref, tpb_ref, wq_ref, wqb_ref, wk_ref,
                     wkb_ref, ln1g_ref, ln1b_ref, qkvw_ref, qkvb_ref, outw_ref,
                     outb_ref, ln2g_ref, ln2b_ref, ff1w_ref, ff1b_ref, ff2w_ref,
                     ff2b_ref, pw_ref, pb_ref, lg_ref, lb_ref, qt_ref, o_ref, g)


def _encoder_one(x2, tpw_ref, tpb_ref, wq_ref, wqb_ref, wk_ref, wkb_ref,
                 ln1g_ref, ln1b_ref, qkvw_ref, qkvb_ref, outw_ref, outb_ref,
                 ln2g_ref, ln2b_ref, ff1w_ref, ff1b_ref, ff2w_ref, ff2b_ref,
                 pw_ref, pb_ref, lg_ref, lb_ref, qt_ref, o_ref, g):
    h = lax.dot_general(x2.astype(jnp.bfloat16),
                        tpw_ref[...].astype(jnp.bfloat16),
                        (((0,), (1,)), ((), ())),
                        preferred_element_type=jnp.float32) + tpb_ref[...]
    q = _dot_t(h, wq_ref[...]) + wqb_ref[...]
    k = _dot_t(h, wk_ref[...]) + wkb_ref[...]
    adj = _softmax(_dot_t(q, k) * (D ** -0.5))
    z = _dot(adj, h)
    for l in range(NL):
        y = _ln(z, ln1g_ref[l], ln1b_ref[l])
        qkv = _dot_t(y, qkvw_ref[l]) + qkvb_ref[l]
        qh, kh, vh = qkv[:, :D], qkv[:, D:2 * D], qkv[:, 2 * D:]
        outs = []
        for hh in range(H):
            sl = slice(hh * DH, (hh + 1) * DH)
            s = _dot_t(qh[:, sl], kh[:, sl]) * (DH ** -0.5)
            outs.append(_dot(_softmax(s), vh[:, sl]))
        o = jnp.concatenate(outs, axis=1)
        z = z + _dot_t(o, outw_ref[l]) + outb_ref[l]
        y2 = _ln(z, ln2g_ref[l], ln2b_ref[l])
        f = jnp.maximum(_dot_t(y2, ff1w_ref[l]) + ff1b_ref[l], 0.0)
        z = z + _dot_t(f, ff2w_ref[l]) + ff2b_ref[l]
    qt = qt_ref[0]
    att = _softmax(_dot_t(qt, z) * (D ** -0.5))
    ctx = _dot(att, z)
    br = _dot_t(ctx, pw_ref[...]) + pb_ref[...]
    o_ref[g] = _ln(br, lg_ref[...], lb_ref[...])


def _gather_kernel(ids_ref, brain_ref, table_ref, out_ref, sem_r):
    b = pl.program_id(0)
    out_ref[0, :NQ, :] = brain_ref[0]
    for s in range(S):
        tok = ids_ref[b, s]
        pltpu.make_async_copy(
            table_ref.at[tok], out_ref.at[0, NQ + s], sem_r).start()
    wait_cp = pltpu.make_async_copy(
        table_ref.at[0], out_ref.at[0, NQ], sem_r)
    for s in range(S):
        wait_cp.wait()


def kernel(bold, input_ids, attention_mask, labels,
           conv1_w, conv1_b, bn1_g, bn1_b, conv2_w, conv2_b, bn2_g, bn2_b,
           tproj_w, tproj_b, wq_w, wq_b, wk_w, wk_b,
           enc_ln1_g, enc_ln1_b, enc_qkv_w, enc_qkv_b, enc_out_w, enc_out_b,
           enc_ln2_g, enc_ln2_b, enc_ff1_w, enc_ff1_b, enc_ff2_w, enc_ff2_b,
           proj_w, proj_b, lnf_g, lnf_b, query_tokens, embed_table):
    f32 = jnp.float32
    w1 = conv1_w.reshape(R, 2, 3)
    w1a = w1[:, 0, :].transpose(1, 0).reshape(3, 1, R)
    w1b = w1[:, 1, :].transpose(1, 0).reshape(3, 1, R)
    b1 = conv1_b.reshape(R, 2)
    b1a, b1b = b1[:, 0].reshape(1, 1, R), b1[:, 1].reshape(1, 1, R)
    g1 = bn1_g.reshape(R, 2)
    g1a, g1b = g1[:, 0].reshape(1, 1, R), g1[:, 1].reshape(1, 1, R)
    h1 = bn1_b.reshape(R, 2)
    h1a, h1b = h1[:, 0].reshape(1, 1, R), h1[:, 1].reshape(1, 1, R)
    w2a = conv2_w[:, 0, :].transpose(1, 0).reshape(3, 1, R)
    w2b = conv2_w[:, 1, :].transpose(1, 0).reshape(3, 1, R)
    c2b = conv2_b.reshape(1, 1, R)
    g2, h2 = bn2_g.reshape(1, 1, R), bn2_b.reshape(1, 1, R)

    x2 = pl.pallas_call(
        _conv_bn_kernel,
        out_shape=jax.ShapeDtypeStruct((B, T, R), f32),
        name="conv_bn",
    )(bold, w1a, w1b, b1a, b1b, g1a, g1b, h1a, h1b, w2a, w2b, c2b, g2, h2)

    full = lambda shape: pl.BlockSpec(shape, lambda b: (0,) * len(shape))
    enc_in_specs = [
        pl.BlockSpec((_EG, T, R), lambda b: (b, 0, 0)),
        full((D, T)), full((1, D)),
        full((D, D)), full((1, D)), full((D, D)), full((1, D)),
        full((NL, D)), full((NL, D)),
        full((NL, 3 * D, D)), full((NL, 3 * D)),
        full((NL, D, D)), full((NL, D)),
        full((NL, D)), full((NL, D)),
        full((NL, FF, D)), full((NL, FF)),
        full((NL, D, FF)), full((NL, D)),
        full((HID, D)), full((1, HID)), full((1, HID)), full((1, HID)),
        full((1, NQ, D)),
    ]
    brain = pl.pallas_call(
        _encoder_kernel,
        grid=(B // _EG,),
        in_specs=enc_in_specs,
        out_specs=pl.BlockSpec((_EG, NQ, HID), lambda b: (b, 0, 0)),
        out_shape=jax.ShapeDtypeStruct((B, NQ, HID), f32),
        compiler_params=pltpu.CompilerParams(
            dimension_semantics=("parallel",),
        ),
        name="encoder",
    )(x2, tproj_w, tproj_b.reshape(1, D),
      wq_w, wq_b.reshape(1, D), wk_w, wk_b.reshape(1, D),
      enc_ln1_g, enc_ln1_b, enc_qkv_w, enc_qkv_b, enc_out_w, enc_out_b,
      enc_ln2_g, enc_ln2_b, enc_ff1_w, enc_ff1_b, enc_ff2_w, enc_ff2_b,
      proj_w, proj_b.reshape(1, HID), lnf_g.reshape(1, HID),
      lnf_b.reshape(1, HID), query_tokens)

    inputs_embeds = pl.pallas_call(
        _gather_kernel,
        grid_spec=pltpu.PrefetchScalarGridSpec(
            num_scalar_prefetch=1,
            grid=(B,),
            in_specs=[pl.BlockSpec((1, NQ, HID), lambda b, ids: (b, 0, 0)),
                      pl.BlockSpec(memory_space=pl.ANY)],
            out_specs=pl.BlockSpec((1, NQ + S, HID), lambda b, ids: (b, 0, 0)),
            scratch_shapes=[pltpu.SemaphoreType.DMA],
        ),
        out_shape=jax.ShapeDtypeStruct((B, NQ + S, HID), f32),
        compiler_params=pltpu.CompilerParams(
            dimension_semantics=("arbitrary",),
            vmem_limit_bytes=50 * 1024 * 1024,
        ),
        name="gather_concat",
    )(input_ids.astype(jnp.int32), brain, embed_table)

    full_mask = jnp.concatenate(
        [jnp.ones((B, NQ), attention_mask.dtype), attention_mask], axis=1)
    full_labels = jnp.concatenate(
        [jnp.full((B, NQ), -100, labels.dtype), labels], axis=1)
    return inputs_embeds, full_mask, full_labels
